# Initial kernel scaffold; baseline (speedup 1.0000x reference)
import jax
import jax.numpy as jnp
from jax.experimental import pallas as pl


def kernel(feat, edge_index_0, edge_index_1, edge_index_2, W1_0, b1_0, W1_1, b1_1, W1_2, b1_2, W2_0, b2_0, W2_1, b2_1, W2_2, b2_2):
    raise NotImplementedError("write your pallas kernel here")


# speedup vs baseline: 124.2969x; 124.2969x over previous
import jax
import jax.numpy as jnp
from jax import lax
from jax.experimental import pallas as pl
from jax.experimental.pallas import tpu as pltpu
from jax.experimental.pallas import tpu_sc as plsc

N = 50000
E = 262144
HW = 8
D = 64
DH = 32
NC = 2
NS = 16
f32 = jnp.float32
i32 = jnp.int32

FL_T = 10
FL_PT = N // FL_T
ZR = 200
CNT_CH = 1000

W1 = 512
NW1 = E // NS // W1
WA = 512
NWA = E // (NC * NS) // WA
_SC_PARAMS = None


def _conv_matrix(W):
  w = W.reshape(3, 3)
  M = jnp.zeros((D, D), f32)
  for a in range(3):
    for b in range(3):
      M = M + w[a, b] * jnp.kron(jnp.eye(HW, k=-(a - 1), dtype=f32),
                                 jnp.eye(HW, k=-(b - 1), dtype=f32))
  return M


def _zero_acc(acc, zbuf, s):
  @pl.when(s < FL_T)
  def _():
    for k in range(FL_PT // ZR):
      pltpu.sync_copy(zbuf, acc.at[pl.ds(s * FL_PT + k * ZR, ZR)])


def _flush_acc(acc, out, c, s):
  @pl.when(s < FL_T)
  def _():
    pltpu.sync_copy(acc.at[pl.ds(s * FL_PT, FL_PT)],
                    out.at[c, pl.ds(s * FL_PT, FL_PT)])


def _init_zbuf(zbuf):
  def body(r, _):
    zbuf[r, pl.ds(0, 16)] = jnp.zeros((16,), f32)
    zbuf[r, pl.ds(16, 16)] = jnp.zeros((16,), f32)
    return _
  lax.fori_loop(0, ZR, body, None)


def _sc1_body(featI, s0, d0, s1, d1, s2, d2,
              S1_0, S1_1, S1_2, c0_out, c1_out, c2_out,
              acc, cnt, sidx, didx, gbuf, ones, zbuf, zvec, sem):
  c = lax.axis_index("c")
  s = lax.axis_index("s")
  srcs = (s0, s1, s2)
  dsts = (d0, d1, d2)
  S1s = (S1_0, S1_1, S1_2)
  couts = (c0_out, c1_out, c2_out)

  def init_ones(j, _):
    ones[pl.ds(j * 16, 16)] = jnp.full((16,), 1.0, f32)
    return _
  lax.fori_loop(0, W1 // 16, init_ones, None)

  def init_zvec(j, _):
    zvec[pl.ds(j * 16, 16)] = jnp.zeros((16,), f32)
    return _
  lax.fori_loop(0, CNT_CH // 16, init_zvec, None)

  _init_zbuf(zbuf)

  for i in range(3):
    plsc.subcore_barrier()
    _zero_acc(acc, zbuf, s)

    @pl.when(jnp.logical_and(c == 0, s < FL_T))
    def _():
      for k in range(FL_PT // CNT_CH):
        pltpu.sync_copy(zvec, cnt.at[pl.ds(s * FL_PT + k * CNT_CH, CNT_CH)])

    plsc.subcore_barrier()

    def window(wi, _):
      base = s * (E // NS) + wi * W1
      pltpu.sync_copy(srcs[i].at[pl.ds(base, W1)], sidx)
      pltpu.sync_copy(dsts[i].at[pl.ds(base, W1)], didx)

      def adj(j, _):
        v = sidx[pl.ds(j * 16, 16)]
        sidx[pl.ds(j * 16, 16)] = v * 2 + c
        return _
      lax.fori_loop(0, W1 // 16, adj, None, unroll=4)

      pltpu.async_copy(featI.at[sidx], gbuf, sem).wait()
      pltpu.sync_copy(gbuf, acc.at[didx], add=True)

      @pl.when(c == 0)
      def _():
        pltpu.sync_copy(ones, cnt.at[didx], add=True)
      return _
    lax.fori_loop(0, NW1, window, None)

    plsc.subcore_barrier()
    _flush_acc(acc, S1s[i], c, s)

    @pl.when(jnp.logical_and(c == 0, s < FL_T))
    def _():
      for k in range(FL_PT // CNT_CH):
        sl = pl.ds(s * FL_PT + k * CNT_CH, CNT_CH)
        pltpu.sync_copy(cnt.at[sl], couts[i].at[sl])


def _sc1(featI, srcs_dsts):
  mesh = plsc.VectorSubcoreMesh(core_axis_name="c", subcore_axis_name="s",
                                num_cores=NC, num_subcores=NS)
  out_type = [jax.ShapeDtypeStruct((NC, N, DH), f32) for _ in range(3)]
  out_type += [jax.ShapeDtypeStruct((N,), f32) for _ in range(3)]
  kern = pl.kernel(
      _sc1_body,
      out_type=out_type,
      mesh=mesh,
      compiler_params=pltpu.CompilerParams(use_tc_tiling_on_sc=False),
      scratch_types=[
          pltpu.VMEM_SHARED((N, DH), f32),
          pltpu.VMEM_SHARED((N,), f32),
          pltpu.VMEM((W1,), i32),
          pltpu.VMEM((W1,), i32),
          pltpu.VMEM((W1, DH), f32),
          pltpu.VMEM((W1,), f32),
          pltpu.VMEM((ZR, DH), f32),
          pltpu.VMEM((CNT_CH,), f32),
          pltpu.SemaphoreType.DMA,
      ],
  )
  return kern(featI, *srcs_dsts)


def _sc3a_body(p0, p1, p2, s0, d0, s1, d1, s2, d2,
               Wh0, Wh1, Wh2,
               sidxA, didxA, bufS, bufD, sem):
  c = lax.axis_index("c")
  s = lax.axis_index("s")
  ps = (p0, p1, p2)
  srcs = (s0, s1, s2)
  dsts = (d0, d1, d2)
  Whs = (Wh0, Wh1, Wh2)
  w = s * NC + c

  for i in range(3):
    def winA(wi, _):
      base = w * (E // (NC * NS)) + wi * WA
      pltpu.sync_copy(srcs[i].at[pl.ds(base, WA)], sidxA)
      pltpu.sync_copy(dsts[i].at[pl.ds(base, WA)], didxA)
      pltpu.async_copy(ps[i].at[sidxA], bufS, sem).wait()
      pltpu.async_copy(ps[i].at[didxA], bufD, sem).wait()

      def add_row(r, _):
        for k in range(D // 16):
          a = bufS[r, pl.ds(k * 16, 16)]
          b = bufD[r, pl.ds(k * 16, 16)]
          bufS[r, pl.ds(k * 16, 16)] = a + b
        return _
      lax.fori_loop(0, WA, add_row, None)
      pltpu.sync_copy(bufS, Whs[i].at[pl.ds(base, WA)])
      return _
    lax.fori_loop(0, NWA, winA, None)


def _sc3a(ps, srcs_dsts):
  mesh = plsc.VectorSubcoreMesh(core_axis_name="c", subcore_axis_name="s",
                                num_cores=NC, num_subcores=NS)
  out_type = [jax.ShapeDtypeStruct((E, D), f32) for _ in range(3)]
  kern = pl.kernel(
      _sc3a_body,
      out_type=out_type,
      mesh=mesh,
      compiler_params=pltpu.CompilerParams(use_tc_tiling_on_sc=False),
      scratch_types=[
          pltpu.VMEM((WA,), i32),
          pltpu.VMEM((WA,), i32),
          pltpu.VMEM((WA, D), f32),
          pltpu.VMEM((WA, D), f32),
          pltpu.SemaphoreType.DMA,
      ],
  )
  return kern(*ps, *srcs_dsts)


def _sc3b_body(q0, q1, q2, s0, d0, s1, d1, s2, d2,
               S2_0, S2_1, S2_2,
               acc, sidx, didx, gbuf, zbuf, sem):
  c = lax.axis_index("c")
  s = lax.axis_index("s")
  qs = (q0, q1, q2)
  srcs = (s0, s1, s2)
  dsts = (d0, d1, d2)
  S2s = (S2_0, S2_1, S2_2)

  _init_zbuf(zbuf)

  for i in range(3):
    plsc.subcore_barrier()
    _zero_acc(acc, zbuf, s)
    plsc.subcore_barrier()

    def winB(wi, _):
      base = s * (E // NS) + wi * W1
      pltpu.sync_copy(srcs[i].at[pl.ds(base, W1)], sidx)
      pltpu.sync_copy(dsts[i].at[pl.ds(base, W1)], didx)

      def adj(j, _):
        v = sidx[pl.ds(j * 16, 16)]
        sidx[pl.ds(j * 16, 16)] = v * 2 + c
        return _
      lax.fori_loop(0, W1 // 16, adj, None, unroll=4)

      pltpu.async_copy(qs[i].at[sidx], gbuf, sem).wait()
      pltpu.sync_copy(gbuf, acc.at[didx], add=True)
      return _
    lax.fori_loop(0, NW1, winB, None)

    plsc.subcore_barrier()
    _flush_acc(acc, S2s[i], c, s)


def _sc3b(qIs, srcs_dsts):
  mesh = plsc.VectorSubcoreMesh(core_axis_name="c", subcore_axis_name="s",
                                num_cores=NC, num_subcores=NS)
  out_type = [jax.ShapeDtypeStruct((NC, N, DH), f32) for _ in range(3)]
  kern = pl.kernel(
      _sc3b_body,
      out_type=out_type,
      mesh=mesh,
      compiler_params=pltpu.CompilerParams(use_tc_tiling_on_sc=False),
      scratch_types=[
          pltpu.VMEM_SHARED((N, DH), f32),
          pltpu.VMEM((W1,), i32),
          pltpu.VMEM((W1,), i32),
          pltpu.VMEM((W1, DH), f32),
          pltpu.VMEM((ZR, DH), f32),
          pltpu.SemaphoreType.DMA,
      ],
  )
  return kern(*qIs, *srcs_dsts)


B = 1000


def _tc2_body(feat_ref, S0_ref, S1_ref, S2_ref, c0_ref, c1_ref, c2_ref,
              M1_ref, M2_ref, b1_ref, b2_ref,
              p0_ref, p1_ref, p2_ref, q0_ref, q1_ref, q2_ref):
  feat = feat_ref[...]
  Ss = (S0_ref[...], S1_ref[...], S2_ref[...])
  cs = (c0_ref[...], c1_ref[...], c2_ref[...])
  M1 = M1_ref[...]
  M2 = M2_ref[...]
  hsum = jnp.zeros((B, D), f32)
  contrib = jnp.zeros((B, 1), f32)
  for i in range(3):
    S = jnp.concatenate([Ss[i][0], Ss[i][1]], axis=1)
    ci = cs[i]
    t = S + ci * feat
    y = S + jax.lax.dot(t, M1[i], precision=jax.lax.Precision.HIGHEST,
                        preferred_element_type=f32) + ci * b1_ref[i]
    hsum = hsum + y / jnp.maximum(ci, 1.0)
    contrib = contrib + (ci > 0.0).astype(f32)
  h1 = hsum / jnp.maximum(contrib, 1.0)
  h1 = jnp.where(h1 >= 0.0, h1, 0.01 * h1)
  pouts = (p0_ref, p1_ref, p2_ref)
  qouts = (q0_ref, q1_ref, q2_ref)
  for i in range(3):
    p = jax.lax.dot(h1, M2[i], precision=jax.lax.Precision.HIGHEST,
                    preferred_element_type=f32) + 0.5 * b2_ref[i]
    pouts[i][...] = p
    qouts[i][...] = h1 + p


def _tc2(feat2, S1s, cnts, M1s, M2s, b1v, b2v):
  grid = (N // B,)
  s_spec = pl.BlockSpec((NC, B, DH), lambda j: (0, j, 0))
  c_spec = pl.BlockSpec((B, 1), lambda j: (j, 0))
  return pl.pallas_call(
      _tc2_body,
      grid=grid,
      in_specs=[
          pl.BlockSpec((B, D), lambda j: (j, 0)),
          s_spec, s_spec, s_spec,
          c_spec, c_spec, c_spec,
          pl.BlockSpec((3, D, D), lambda j: (0, 0, 0)),
          pl.BlockSpec((3, D, D), lambda j: (0, 0, 0)),
          pl.BlockSpec(memory_space=pltpu.SMEM),
          pl.BlockSpec(memory_space=pltpu.SMEM),
      ],
      out_specs=[pl.BlockSpec((B, D), lambda j: (j, 0)) for _ in range(6)],
      out_shape=[jax.ShapeDtypeStruct((N, D), f32) for _ in range(6)],
  )(feat2, *S1s, *cnts, M1s, M2s, b1v, b2v)


def _tc4_body(S0_ref, S1_ref, S2_ref, c0_ref, c1_ref, c2_ref,
              p0_ref, p1_ref, p2_ref, h2_ref):
  Ss = (S0_ref[...], S1_ref[...], S2_ref[...])
  cs = (c0_ref[...], c1_ref[...], c2_ref[...])
  pblk = (p0_ref[...], p1_ref[...], p2_ref[...])
  hsum = jnp.zeros((B, D), f32)
  contrib = jnp.zeros((B, 1), f32)
  for i in range(3):
    S = jnp.concatenate([Ss[i][0], Ss[i][1]], axis=1)
    ci = cs[i]
    msum = S + ci * pblk[i]
    hsum = hsum + msum / jnp.maximum(ci, 1.0)
    contrib = contrib + (ci > 0.0).astype(f32)
  h2_ref[...] = hsum / jnp.maximum(contrib, 1.0)


def _tc4(S2s, cnts, ps):
  grid = (N // B,)
  s_spec = pl.BlockSpec((NC, B, DH), lambda j: (0, j, 0))
  c_spec = pl.BlockSpec((B, 1), lambda j: (j, 0))
  p_spec = pl.BlockSpec((B, D), lambda j: (j, 0))
  return pl.pallas_call(
      _tc4_body,
      grid=grid,
      in_specs=[s_spec, s_spec, s_spec, c_spec, c_spec, c_spec,
                p_spec, p_spec, p_spec],
      out_specs=pl.BlockSpec((B, D), lambda j: (j, 0)),
      out_shape=jax.ShapeDtypeStruct((N, D), f32),
  )(*S2s, *cnts, *ps)


def kernel(feat, edge_index_0, edge_index_1, edge_index_2,
           W1_0, b1_0, W1_1, b1_1, W1_2, b1_2,
           W2_0, b2_0, W2_1, b2_1, W2_2, b2_2):
  feat2 = feat.reshape(N, D)
  featI = feat2.reshape(NC * N, DH)

  srcs_dsts = []
  for ei in (edge_index_0, edge_index_1, edge_index_2):
    ei = jnp.asarray(ei, i32)
    srcs_dsts.append(ei[0])
    srcs_dsts.append(ei[1])

  M1s = jnp.stack([_conv_matrix(W1_0), _conv_matrix(W1_1), _conv_matrix(W1_2)])
  M2s = jnp.stack([_conv_matrix(W2_0), _conv_matrix(W2_1), _conv_matrix(W2_2)])
  b1v = jnp.stack([b1_0[0], b1_1[0], b1_2[0]])
  b2v = jnp.stack([b2_0[0], b2_1[0], b2_2[0]])

  S1_0, S1_1, S1_2, c0, c1, c2 = _sc1(featI, srcs_dsts)
  cnts = [c.reshape(N, 1) for c in (c0, c1, c2)]

  p0, p1, p2, q0, q1, q2 = _tc2(feat2, (S1_0, S1_1, S1_2), cnts,
                                M1s, M2s, b1v, b2v)
  qIs = [q.reshape(NC * N, DH) for q in (q0, q1, q2)]

  Wh0, Wh1, Wh2 = _sc3a((p0, p1, p2), srcs_dsts)
  S2s = _sc3b(qIs, srcs_dsts)

  h2 = _tc4(S2s, cnts, (p0, p1, p2))

  return (h2.reshape(N, 1, HW, HW),
          Wh0.reshape(E, 1, HW, HW),
          Wh1.reshape(E, 1, HW, HW),
          Wh2.reshape(E, 1, HW, HW))
